# baseline (speedup 1.0000x reference)
import functools

import jax
import jax.numpy as jnp
from jax import lax
from jax.experimental import pallas as pl
from jax.experimental.pallas import tpu as pltpu
from jax.experimental.pallas import tpu_sc as plsc

_N_TABLES = 26
_VOCAB = 100000
_D = 32
_B = 16384
_NC = 2
_NS = 16
_NW = _NC * _NS
_Q = _B // 4
_QR = _Q // 128


def _body(x_hbm, tab_hbm, out_hbm, row_v, x_v, res_a, res_b, osem):
    wid = lax.axis_index("s") * _NC + lax.axis_index("c")
    dt = wid // 8
    r = wid % 8
    res = (res_a, res_b)
    pending = [None, None]

    pltpu.sync_copy(x_hbm, x_v)
    step = 0
    for t in range(_N_TABLES):
        pltpu.sync_copy(tab_hbm.at[t, wid], row_v)
        for h in range(4):
            slot = step % 2
            buf = res[slot]
            if pending[slot] is not None:
                pending[slot].wait()

            def sel(row, carry):
                for c in range(8):
                    xv = x_v[pl.ds(h * _Q + row * 128 + c * 16, 16)]
                    buf[row, pl.ds(c * 16, 16)] = plsc.load_gather(
                        row_v, [xv])
                return carry

            lax.fori_loop(0, _QR, sel, 0, unroll=2)

            pending[slot] = pltpu.async_copy(
                buf, out_hbm.at[t, dt, pl.ds(h * _QR, _QR), r], osem)
            step += 1
    for cp in pending:
        if cp is not None:
            cp.wait()


def kernel(x, tables):
    tab_t = jnp.transpose(tables, (0, 2, 1))
    run = pl.kernel(
        _body,
        out_type=jax.ShapeDtypeStruct((_N_TABLES, 4, _B // 128, 8, 128),
                                      jnp.float32),
        mesh=plsc.VectorSubcoreMesh(
            core_axis_name="c", subcore_axis_name="s",
            num_cores=_NC, num_subcores=_NS),
        scratch_types=[
            pltpu.VMEM((_VOCAB,), jnp.float32),
            pltpu.VMEM((_B,), jnp.int32),
            pltpu.VMEM((_QR, 128), jnp.float32),
            pltpu.VMEM((_QR, 128), jnp.float32),
            pltpu.SemaphoreType.DMA,
        ],
        compiler_params=pltpu.CompilerParams(
            use_tc_tiling_on_sc=True, needs_layout_passes=False),
    )
    out5d = run(x.astype(jnp.int32), tab_t)
    out = out5d.transpose(2, 4, 0, 1, 3).reshape(_B, _N_TABLES, _D)
    return out


# speedup vs baseline: 1.4394x; 1.4394x over previous
import functools

import jax
import jax.numpy as jnp
from jax import lax
from jax.experimental import pallas as pl
from jax.experimental.pallas import tpu as pltpu
from jax.experimental.pallas import tpu_sc as plsc

_N_TABLES = 26
_VOCAB = 100000
_D = 32
_B = 16384
_NC = 2
_NS = 16
_NW = _NC * _NS
_Q = _B // 4
_QR = _Q // 128


def _body(x_hbm, tab_hbm, out_hbm, row_v, x_v, res_a, res_b, osem):
    wid = lax.axis_index("s") * _NC + lax.axis_index("c")
    dt = wid // 8
    r = wid % 8
    res = (res_a, res_b)
    pending = [None, None]

    pltpu.sync_copy(x_hbm, x_v)
    step = 0
    for t in range(_N_TABLES):
        pltpu.sync_copy(tab_hbm.at[t, wid], row_v)
        for h in range(4):
            slot = step % 2
            buf = res[slot]
            if pending[slot] is not None:
                pending[slot].wait()

            def sel(row, carry):
                for c in range(8):
                    xv = x_v[pl.ds(h * _Q + row * 128 + c * 16, 16)]
                    buf[row, pl.ds(c * 16, 16)] = plsc.load_gather(
                        row_v, [xv])
                return carry

            lax.fori_loop(0, _QR, sel, 0)

            pending[slot] = pltpu.async_copy(
                buf, out_hbm.at[t, dt, pl.ds(h * _QR, _QR), r], osem)
            step += 1
    for cp in pending:
        if cp is not None:
            cp.wait()


def kernel(x, tables):
    tab_t = jnp.transpose(tables, (0, 2, 1))
    run = pl.kernel(
        _body,
        out_type=jax.ShapeDtypeStruct((_N_TABLES, 4, _B // 128, 8, 128),
                                      jnp.float32),
        mesh=plsc.VectorSubcoreMesh(
            core_axis_name="c", subcore_axis_name="s",
            num_cores=_NC, num_subcores=_NS),
        scratch_types=[
            pltpu.VMEM((_VOCAB,), jnp.float32),
            pltpu.VMEM((_B,), jnp.int32),
            pltpu.VMEM((_QR, 128), jnp.float32),
            pltpu.VMEM((_QR, 128), jnp.float32),
            pltpu.SemaphoreType.DMA,
        ],
        compiler_params=pltpu.CompilerParams(
            use_tc_tiling_on_sc=True, needs_layout_passes=False),
    )
    out5d = run(x.astype(jnp.int32), tab_t)
    out = out5d.transpose(2, 4, 0, 1, 3).reshape(_B, _N_TABLES, _D)
    return out
